# baseline (speedup 1.0000x reference)
import functools

import jax
import jax.numpy as jnp
from jax import lax
from jax.experimental import pallas as pl
from jax.experimental.pallas import tpu as pltpu
from jax.experimental.pallas import tpu_sc as plsc

VOCAB = 100000
EMBED_DIM = 64
BATCH = 16384
_WIDE = 2 * EMBED_DIM

_info = plsc.get_sparse_core_info()
_NC, _NS = _info.num_cores, _info.num_subcores
_NW = _NC * _NS
_B_PER_W = BATCH // _NW
_CHUNK = 128
_N_CHUNKS = _B_PER_W // _CHUNK

_TBLK = 12800
_TGRID = -(-VOCAB // _TBLK)

_mesh = plsc.VectorSubcoreMesh(core_axis_name="c", subcore_axis_name="s")


def _transpose_body(x_ref, o_ref):
    t = x_ref[...].T
    o_ref[...] = jnp.concatenate([t, t], axis=1)


_tc_transpose = pl.pallas_call(
    _transpose_body,
    grid=(_TGRID,),
    in_specs=[pl.BlockSpec((EMBED_DIM, _TBLK), lambda i: (0, i))],
    out_specs=pl.BlockSpec((_TBLK, _WIDE), lambda i: (i, 0)),
    out_shape=jax.ShapeDtypeStruct((VOCAB, _WIDE), jnp.float32),
)


@functools.partial(
    pl.kernel,
    mesh=_mesh,
    out_type=jax.ShapeDtypeStruct((BATCH, _WIDE), jnp.float32),
    scratch_types=[
        pltpu.VMEM((_B_PER_W,), jnp.int32),
        pltpu.VMEM((_B_PER_W, _WIDE), jnp.float32),
        pltpu.SemaphoreType.DMA,
    ],
)
def _embedding_gather(table_hbm, idx_hbm, out_hbm, idx_v, rows_v, sem):
    wid = lax.axis_index("s") * _NC + lax.axis_index("c")
    base = wid * _B_PER_W
    pltpu.sync_copy(idx_hbm.at[pl.ds(base, _B_PER_W)], idx_v)
    copies = []
    for j in range(_N_CHUNKS):
        copies.append(
            pltpu.async_copy(
                table_hbm.at[idx_v.at[pl.ds(j * _CHUNK, _CHUNK)]],
                rows_v.at[pl.ds(j * _CHUNK, _CHUNK)],
                sem,
            )
        )
    for c in copies:
        c.wait()
    pltpu.sync_copy(rows_v, out_hbm.at[pl.ds(base, _B_PER_W)])


def kernel(drug_ids, table):
    table_wide = _tc_transpose(table.T)
    rows = _embedding_gather(table_wide, drug_ids.astype(jnp.int32))
    return rows[:, :EMBED_DIM]


# speedup vs baseline: 1.2459x; 1.2459x over previous
import functools

import jax
import jax.numpy as jnp
from jax import lax
from jax.experimental import pallas as pl
from jax.experimental.pallas import tpu as pltpu
from jax.experimental.pallas import tpu_sc as plsc

VOCAB = 100000
EMBED_DIM = 64
BATCH = 16384

_info = plsc.get_sparse_core_info()
_NC, _NS = _info.num_cores, _info.num_subcores
_NW = _NC * _NS
_B_PER_W = BATCH // _NW

_TBLK = 25600
_TGRID = VOCAB // _TBLK

_mesh = plsc.VectorSubcoreMesh(core_axis_name="c", subcore_axis_name="s")


def _transpose_body(x_ref, o_ref):
    o_ref[...] = x_ref[...].T


_tc_transpose = pl.pallas_call(
    _transpose_body,
    grid=(_TGRID,),
    in_specs=[pl.BlockSpec((EMBED_DIM, _TBLK), lambda i: (0, i))],
    out_specs=pl.BlockSpec((_TBLK, EMBED_DIM), lambda i: (i, 0)),
    out_shape=jax.ShapeDtypeStruct((VOCAB, EMBED_DIM), jnp.float32),
)


@functools.partial(
    pl.kernel,
    mesh=_mesh,
    out_type=jax.ShapeDtypeStruct((BATCH, EMBED_DIM), jnp.float32),
    scratch_types=[
        pltpu.VMEM((_B_PER_W,), jnp.int32),
        pltpu.VMEM((_B_PER_W, EMBED_DIM), jnp.float32),
        pltpu.SemaphoreType.DMA,
    ],
)
def _embedding_gather(table_hbm, idx_hbm, out_hbm, idx_v, rows_v, sem):
    wid = lax.axis_index("s") * _NC + lax.axis_index("c")
    base = wid * _B_PER_W
    pltpu.sync_copy(idx_hbm.at[pl.ds(base, _B_PER_W)], idx_v)

    def body(g, _):
        vec = idx_v[pl.ds(g * 16, 16)]
        for l in range(16):
            pltpu.async_copy(
                table_hbm.at[pl.ds(vec[l], 1)],
                rows_v.at[pl.ds(g * 16 + l, 1)],
                sem,
            )
        return ()

    lax.fori_loop(0, _B_PER_W // 16, body, ())
    pltpu.make_async_copy(
        table_hbm.at[pl.ds(0, _B_PER_W)], rows_v, sem
    ).wait()
    pltpu.sync_copy(rows_v, out_hbm.at[pl.ds(base, _B_PER_W)])


def kernel(drug_ids, table):
    table_rows = _tc_transpose(table.T)
    return _embedding_gather(table_rows, drug_ids.astype(jnp.int32))
